# baseline (speedup 1.0000x reference)
import jax
import jax.numpy as jnp
from jax import lax
from jax.experimental import pallas as pl
from jax.experimental.pallas import tpu as pltpu
from jax.experimental.pallas import tpu_sc as plsc

_NC = 2
_NS = 16
_NW = _NC * _NS
_L = 16
_D = 32
_FB = 4
_B = 16384
_BPW = _B // _NW
_NCHUNK = 4
_CH = _BPW // _NCHUNK
_V = 1000000
_REG = 0.001


def _sc_body(rows_hbm, cols_hbm, p3, q3, preds_hbm, regs_hbm,
             idx_r, idx_c, pe, qe, out_v, reg_v, sem_p, sem_q):
    wid = lax.axis_index("s") * _NC + lax.axis_index("c")
    pltpu.sync_copy(rows_hbm.at[wid], idx_r)
    pltpu.sync_copy(cols_hbm.at[wid], idx_c)

    def start_f(f):
        for c in range(_NCHUNK):
            pltpu.async_copy(
                p3.at[f].at[idx_r.at[c]], pe.at[f].at[c], sem_p)
            pltpu.async_copy(
                q3.at[f].at[idx_c.at[c]], qe.at[f].at[c], sem_q)

    def wait_f(f):
        for c in range(_NCHUNK):
            pltpu.make_async_copy(
                p3.at[f].at[idx_r.at[c]], pe.at[f].at[c], sem_p).wait()
            pltpu.make_async_copy(
                q3.at[f].at[idx_c.at[c]], qe.at[f].at[c], sem_q).wait()

    start_f(0)

    def gather_step(f, _):
        start_f(f)
        wait_f(f - 1)
        return 0

    lax.fori_loop(1, _D, gather_step, 0)
    wait_f(_D - 1)

    def group(g, carry):
        accp, accq = carry
        c = g // 8
        b = (g % 8) * _L
        acc = jnp.zeros((_L,), jnp.float32)
        for f in range(_D):
            pv = pe[f, c, pl.ds(b, _L)]
            qv = qe[f, c, pl.ds(b, _L)]
            acc = acc + pv * qv
            accp = accp + pv * pv
            accq = accq + qv * qv
        out_v[pl.ds(g * _L, _L)] = acc
        return accp, accq

    zero = jnp.zeros((_L,), jnp.float32)
    accp, accq = lax.fori_loop(0, _BPW // _L, group, (zero, zero))
    reg_v[0] = accp * _REG
    reg_v[1] = accq * _REG

    pltpu.sync_copy(out_v, preds_hbm.at[wid])
    pltpu.sync_copy(reg_v, regs_hbm.at[wid])


@jax.jit
def kernel(rows, cols, ratval, P, Q):
    del ratval
    rows3 = rows.reshape(_NW, _NCHUNK, _CH)
    cols3 = cols.reshape(_NW, _NCHUNK, _CH)
    p3, q3 = lax.optimization_barrier(
        (jnp.swapaxes(P, 0, 1), jnp.swapaxes(Q, 0, 1)))
    mesh = plsc.VectorSubcoreMesh(core_axis_name="c", subcore_axis_name="s")
    run = pl.kernel(
        _sc_body,
        out_type=[
            jax.ShapeDtypeStruct((_NW, _BPW), jnp.float32),
            jax.ShapeDtypeStruct((_NW, 2, _L), jnp.float32),
        ],
        mesh=mesh,
        compiler_params=pltpu.CompilerParams(
            needs_layout_passes=False,
            use_tc_tiling_on_sc=False,
        ),
        scratch_types=[
            pltpu.VMEM((_NCHUNK, _CH), jnp.int32),
            pltpu.VMEM((_NCHUNK, _CH), jnp.int32),
            pltpu.VMEM((_D, _NCHUNK, _CH), jnp.float32),
            pltpu.VMEM((_D, _NCHUNK, _CH), jnp.float32),
            pltpu.VMEM((_BPW,), jnp.float32),
            pltpu.VMEM((2, _L), jnp.float32),
            pltpu.SemaphoreType.DMA,
            pltpu.SemaphoreType.DMA,
        ],
    )
    preds, regs = run(rows3, cols3, p3, q3)
    preds_rat = preds.reshape(_B, 1)
    ues_reg = jnp.sum(regs[:, 0, :])
    uis_rat_reg = jnp.sum(regs[:, 1, :])
    return (preds_rat, ues_reg, uis_rat_reg)


# speedup vs baseline: 5.6490x; 5.6490x over previous
import functools

import jax
import jax.numpy as jnp
from jax import lax
from jax.experimental import pallas as pl
from jax.experimental.pallas import tpu as pltpu
from jax.experimental.pallas import tpu_sc as plsc

_NC = 2
_NS = 16
_NW = _NC * _NS
_L = 16
_D = 32
_B = 16384
_BPW = _B // _NW
_NCHUNK = 4
_CH = _BPW // _NCHUNK
_REG = 0.001


def _sc_body(rows_hbm, cols_hbm, p_hbm, q_hbm, preds_hbm, regs_hbm,
             idx_r, idx_c, pe, qe, out_v, reg_v, sem_p, sem_q):
    wid = lax.axis_index("s") * _NC + lax.axis_index("c")
    pltpu.sync_copy(rows_hbm.at[wid], idx_r)
    pltpu.sync_copy(cols_hbm.at[wid], idx_c)

    copies = []
    for c in range(_NCHUNK):
        copies.append(pltpu.async_copy(
            p_hbm.at[idx_r.at[c]], pe.at[pl.ds(c * _CH, _CH)], sem_p))
        copies.append(pltpu.async_copy(
            q_hbm.at[idx_c.at[c]], qe.at[pl.ds(c * _CH, _CH)], sem_q))
    for cp in copies:
        cp.wait()

    iota = lax.iota(jnp.int32, _L)

    def group(g, carry):
        accp, accq = carry
        row0 = g * _L
        rowidx = row0 + iota
        acc = jnp.zeros((_L,), jnp.float32)
        for j in range(_D):
            colidx = jnp.full((_L,), j, jnp.int32)
            pv = plsc.load_gather(pe, [rowidx, colidx])
            qv = plsc.load_gather(qe, [rowidx, colidx])
            acc = acc + pv * qv
            accp = accp + pv * pv
            accq = accq + qv * qv
        out_v[pl.ds(row0, _L)] = acc
        return accp, accq

    zero = jnp.zeros((_L,), jnp.float32)
    accp, accq = lax.fori_loop(0, _BPW // _L, group, (zero, zero))
    reg_v[0] = accp * _REG
    reg_v[1] = accq * _REG

    pltpu.sync_copy(out_v, preds_hbm.at[wid])
    pltpu.sync_copy(reg_v, regs_hbm.at[wid])


@jax.jit
def kernel(rows, cols, ratval, P, Q):
    del ratval
    rows3 = rows.reshape(_NW, _NCHUNK, _CH)
    cols3 = cols.reshape(_NW, _NCHUNK, _CH)
    mesh = plsc.VectorSubcoreMesh(core_axis_name="c", subcore_axis_name="s")
    run = pl.kernel(
        _sc_body,
        out_type=[
            jax.ShapeDtypeStruct((_NW, _BPW), jnp.float32),
            jax.ShapeDtypeStruct((_NW, 2, _L), jnp.float32),
        ],
        mesh=mesh,
        compiler_params=pltpu.CompilerParams(
            needs_layout_passes=False,
            use_tc_tiling_on_sc=False,
        ),
        scratch_types=[
            pltpu.VMEM((_NCHUNK, _CH), jnp.int32),
            pltpu.VMEM((_NCHUNK, _CH), jnp.int32),
            pltpu.VMEM((_BPW, _D), jnp.float32),
            pltpu.VMEM((_BPW, _D), jnp.float32),
            pltpu.VMEM((_BPW,), jnp.float32),
            pltpu.VMEM((2, _L), jnp.float32),
            pltpu.SemaphoreType.DMA,
            pltpu.SemaphoreType.DMA,
        ],
    )
    preds, regs = run(rows3, cols3, P, Q)
    preds_rat = preds.reshape(_B, 1)
    ues_reg = jnp.sum(regs[:, 0, :])
    uis_rat_reg = jnp.sum(regs[:, 1, :])
    return (preds_rat, ues_reg, uis_rat_reg)
